# Initial kernel scaffold; baseline (speedup 1.0000x reference)
import jax
import jax.numpy as jnp
from jax.experimental import pallas as pl


def kernel(feature_out, labels, indexes):
    raise NotImplementedError("write your pallas kernel here")


# speedup vs baseline: 20.9134x; 20.9134x over previous
import jax
import jax.numpy as jnp
from jax import lax
from jax.experimental import pallas as pl
from jax.experimental.pallas import tpu as pltpu

N_CLASSES = 5
K = 32 * N_CLASSES
D = 192
HW = 384 * 384
B = 2
PB = 4096
NB = HW // PB
IGNORE_LB = 255


def _i0():
    return jnp.int32(0)


def _segsum_body(lab_ref, idx_ref, feat_ref, sums_ref, counts_ref):
    j = pl.program_id(1)

    @pl.when(j == 0)
    def _init():
        sums_ref[...] = jnp.zeros_like(sums_ref)
        counts_ref[...] = jnp.zeros_like(counts_ref)

    lab = lab_ref[0]
    idx = idx_ref[0]
    k2 = jnp.where(lab == IGNORE_LB, 0, lab * 32 + idx)
    cols = lax.broadcasted_iota(jnp.int32, (PB, K), 1)
    onehot = (k2 == cols).astype(jnp.float32)
    feat = feat_ref[0]
    acc = jnp.dot(feat, onehot, preferred_element_type=jnp.float32)
    sums_ref[...] += acc[None]
    counts_ref[...] += jnp.sum(onehot, axis=0, keepdims=True)[None]


def _finalize_body(sums_ref, counts_ref, out_ref):
    f32 = jnp.float32
    KK = (K, K)
    row = lax.broadcasted_iota(jnp.int32, KK, 0)
    col = lax.broadcasted_iota(jnp.int32, KK, 1)
    eye = row == col
    ks = lax.broadcasted_iota(jnp.int32, (1, K), 1)

    def _col(x, zero):
        return jnp.sum(jnp.where(eye, jnp.broadcast_to(x, KK), zero),
                       axis=1, keepdims=True, dtype=x.dtype)

    total = f32(0.0)
    n_valid = f32(0.0)

    for b in range(B):
        c2 = counts_ref[b]
        s2 = sums_ref[b]
        mx = jnp.max(jnp.where(c2 > 0, jnp.bitwise_and(ks, 31), 0))
        mx_safe = jnp.maximum(mx, 1)
        keyed = mx * jnp.right_shift(row, 5) + jnp.bitwise_and(row, 31)
        remap = (keyed == col).astype(f32)
        counts = jnp.dot(c2, remap, preferred_element_type=f32)
        sums = jnp.dot(s2, remap, preferred_element_type=f32)
        means = sums / jnp.maximum(counts, 1.0)

        present = counts > 0
        n_present = jnp.sum(present.astype(f32))

        P = jnp.zeros(KK, f32)
        for db in range(D // 8):
            md = means[db * 8:(db + 1) * 8, :]
            diff = jnp.abs(md[:, :, None] - md[:, None, :])
            P = P + jnp.sum(diff, axis=0)
        P = P * f32(1.0 / D)

        pk = jnp.where(present, ks, -1)
        k_last = jnp.max(pk)
        k_prev = jnp.max(jnp.where(ks == k_last, -1, pk))
        mxf = mx_safe.astype(f32)
        cls_f = jnp.floor((ks.astype(f32) - 1.0) / mxf)
        eff = jnp.where(ks == k_last,
                        jnp.floor((k_prev.astype(f32) - 1.0) / mxf),
                        cls_f)
        included = present & (counts >= 2.0) & (ks >= 1)
        incval = jnp.where(included, ks, -1)
        inc_col = _col(incval, 0)
        strict = row < col
        prev_idx = jnp.max(jnp.where(strict, jnp.broadcast_to(inc_col, KK), -1),
                           axis=0, keepdims=True)
        pidx = jnp.maximum(prev_idx, 0)
        eff_col = _col(eff, f32(0.0))
        gmat = row == jnp.broadcast_to(pidx, KK)
        prev_cls = jnp.sum(jnp.where(gmat, jnp.broadcast_to(eff_col, KK), 0.0),
                           axis=0, keepdims=True)
        prev_cls = jnp.where(prev_idx < 0, f32(-1e9), prev_cls)
        new_group = included & (eff != prev_cls)
        ng_col = _col(new_group.astype(f32), f32(0.0))
        lower = row <= col
        cums = jnp.sum(jnp.where(lower, jnp.broadcast_to(ng_col, KK), 0.0),
                       axis=0, keepdims=True)
        gid = jnp.where(included, cums - 1.0, f32(-1.0))
        memb = (row.astype(f32) == jnp.broadcast_to(gid, KK)).astype(f32)
        sizes = jnp.sum(memb, axis=1, keepdims=True)
        mp = jnp.dot(memb, P, preferred_element_type=f32)
        num = jnp.sum(mp * memb, axis=1, keepdims=True)
        ret = num / (jnp.maximum(sizes, 1.0) ** 2)
        ret = jnp.where(ret < 1.0, 0.5 * ret * ret, ret - 0.5)
        validg = (sizes > 0.0) & (n_present > 1.0)
        total = total + jnp.sum(jnp.where(validg, ret, 0.0))
        n_valid = n_valid + jnp.sum(validg.astype(f32))

    loss = jnp.where(n_valid > 0.0, total / jnp.maximum(n_valid, 1.0), 0.0)
    out_ref[...] = jnp.broadcast_to(loss * f32(1.0 / B), (1, 1))


def kernel(feature_out, labels, indexes):
    lab = labels.astype(jnp.int32).reshape(B, HW, 1)
    idx = indexes.astype(jnp.int32).reshape(B, HW, 1)
    feat = feature_out.reshape(B, D, HW)
    sums, counts = pl.pallas_call(
        _segsum_body,
        grid=(B, NB),
        in_specs=[
            pl.BlockSpec((1, PB, 1), lambda b, j: (b, j, _i0())),
            pl.BlockSpec((1, PB, 1), lambda b, j: (b, j, _i0())),
            pl.BlockSpec((1, D, PB), lambda b, j: (b, _i0(), j)),
        ],
        out_specs=[
            pl.BlockSpec((1, D, K), lambda b, j: (b, _i0(), _i0())),
            pl.BlockSpec((1, 1, K), lambda b, j: (b, _i0(), _i0())),
        ],
        out_shape=[
            jax.ShapeDtypeStruct((B, D, K), jnp.float32),
            jax.ShapeDtypeStruct((B, 1, K), jnp.float32),
        ],
        compiler_params=pltpu.CompilerParams(
            dimension_semantics=("arbitrary", "arbitrary")),
    )(lab, idx, feat)
    loss = pl.pallas_call(
        _finalize_body,
        out_shape=jax.ShapeDtypeStruct((1, 1), jnp.float32),
    )(sums, counts)
    return loss.reshape(1)


# speedup vs baseline: 37.2443x; 1.7809x over previous
import jax
import jax.numpy as jnp
from jax import lax
from jax.experimental import pallas as pl
from jax.experimental.pallas import tpu as pltpu

N_CLASSES = 5
K = 32 * N_CLASSES
D = 192
HW = 384 * 384
B = 2
PB = 4096
NB = HW // PB
IGNORE_LB = 255


def _i0():
    return jnp.int32(0)


def _segsum_body(lab_ref, idx_ref, feat_ref, sums_ref, counts_ref):
    j = pl.program_id(1)

    @pl.when(j == 0)
    def _init():
        sums_ref[...] = jnp.zeros_like(sums_ref)
        counts_ref[...] = jnp.zeros_like(counts_ref)

    lab = lab_ref[0]
    idx = idx_ref[0]
    k2 = jnp.where(lab == IGNORE_LB, 0, lab * 32 + idx)
    rows = lax.broadcasted_iota(jnp.int32, (K, PB), 0)
    oh_t = (rows == jnp.broadcast_to(k2, (K, PB))).astype(jnp.bfloat16)
    feat = feat_ref[0]
    hi = feat.astype(jnp.bfloat16)
    lo = (feat - hi.astype(jnp.float32)).astype(jnp.bfloat16)
    dn = (((1,), (1,)), ((), ()))
    acc = (lax.dot_general(hi, oh_t, dn, preferred_element_type=jnp.float32)
           + lax.dot_general(lo, oh_t, dn, preferred_element_type=jnp.float32))
    sums_ref[...] += acc[None]
    ones = jnp.ones((8, PB), jnp.bfloat16)
    cnt = lax.dot_general(ones, oh_t, dn, preferred_element_type=jnp.float32)
    counts_ref[...] += cnt[0:1][None]


def _finalize_body(sums_ref, counts_ref, out_ref):
    f32 = jnp.float32
    KK = (K, K)
    row = lax.broadcasted_iota(jnp.int32, KK, 0)
    col = lax.broadcasted_iota(jnp.int32, KK, 1)
    eye = row == col
    ks = lax.broadcasted_iota(jnp.int32, (1, K), 1)

    def _col(x, zero):
        return jnp.sum(jnp.where(eye, jnp.broadcast_to(x, KK), zero),
                       axis=1, keepdims=True, dtype=x.dtype)

    total = f32(0.0)
    n_valid = f32(0.0)

    for b in range(B):
        c2 = counts_ref[b]
        s2 = sums_ref[b]
        mx = jnp.max(jnp.where(c2 > 0, jnp.bitwise_and(ks, 31), 0))
        mx_safe = jnp.maximum(mx, 1)
        keyed = mx * jnp.right_shift(row, 5) + jnp.bitwise_and(row, 31)
        remap = (keyed == col).astype(f32)
        counts = jnp.dot(c2, remap, preferred_element_type=f32)
        sums = jnp.dot(s2, remap, preferred_element_type=f32)
        means = sums / jnp.maximum(counts, 1.0)

        present = counts > 0
        n_present = jnp.sum(present.astype(f32))

        P = jnp.zeros(KK, f32)
        for db in range(D // 8):
            md = means[db * 8:(db + 1) * 8, :]
            diff = jnp.abs(md[:, :, None] - md[:, None, :])
            P = P + jnp.sum(diff, axis=0)
        P = P * f32(1.0 / D)

        pk = jnp.where(present, ks, -1)
        k_last = jnp.max(pk)
        k_prev = jnp.max(jnp.where(ks == k_last, -1, pk))
        mxf = mx_safe.astype(f32)
        cls_f = jnp.floor((ks.astype(f32) - 1.0) / mxf)
        eff = jnp.where(ks == k_last,
                        jnp.floor((k_prev.astype(f32) - 1.0) / mxf),
                        cls_f)
        included = present & (counts >= 2.0) & (ks >= 1)
        incval = jnp.where(included, ks, -1)
        inc_col = _col(incval, 0)
        strict = row < col
        prev_idx = jnp.max(jnp.where(strict, jnp.broadcast_to(inc_col, KK), -1),
                           axis=0, keepdims=True)
        pidx = jnp.maximum(prev_idx, 0)
        eff_col = _col(eff, f32(0.0))
        gmat = row == jnp.broadcast_to(pidx, KK)
        prev_cls = jnp.sum(jnp.where(gmat, jnp.broadcast_to(eff_col, KK), 0.0),
                           axis=0, keepdims=True)
        prev_cls = jnp.where(prev_idx < 0, f32(-1e9), prev_cls)
        new_group = included & (eff != prev_cls)
        ng_col = _col(new_group.astype(f32), f32(0.0))
        lower = row <= col
        cums = jnp.sum(jnp.where(lower, jnp.broadcast_to(ng_col, KK), 0.0),
                       axis=0, keepdims=True)
        gid = jnp.where(included, cums - 1.0, f32(-1.0))
        memb = (row.astype(f32) == jnp.broadcast_to(gid, KK)).astype(f32)
        sizes = jnp.sum(memb, axis=1, keepdims=True)
        mp = jnp.dot(memb, P, preferred_element_type=f32)
        num = jnp.sum(mp * memb, axis=1, keepdims=True)
        ret = num / (jnp.maximum(sizes, 1.0) ** 2)
        ret = jnp.where(ret < 1.0, 0.5 * ret * ret, ret - 0.5)
        validg = (sizes > 0.0) & (n_present > 1.0)
        total = total + jnp.sum(jnp.where(validg, ret, 0.0))
        n_valid = n_valid + jnp.sum(validg.astype(f32))

    loss = jnp.where(n_valid > 0.0, total / jnp.maximum(n_valid, 1.0), 0.0)
    out_ref[...] = jnp.broadcast_to(loss * f32(1.0 / B), (1, 1))


def kernel(feature_out, labels, indexes):
    lab = labels.astype(jnp.int32).reshape(B, 1, HW)
    idx = indexes.astype(jnp.int32).reshape(B, 1, HW)
    feat = feature_out.reshape(B, D, HW)
    sums, counts = pl.pallas_call(
        _segsum_body,
        grid=(B, NB),
        in_specs=[
            pl.BlockSpec((1, 1, PB), lambda b, j: (b, _i0(), j)),
            pl.BlockSpec((1, 1, PB), lambda b, j: (b, _i0(), j)),
            pl.BlockSpec((1, D, PB), lambda b, j: (b, _i0(), j)),
        ],
        out_specs=[
            pl.BlockSpec((1, D, K), lambda b, j: (b, _i0(), _i0())),
            pl.BlockSpec((1, 1, K), lambda b, j: (b, _i0(), _i0())),
        ],
        out_shape=[
            jax.ShapeDtypeStruct((B, D, K), jnp.float32),
            jax.ShapeDtypeStruct((B, 1, K), jnp.float32),
        ],
        compiler_params=pltpu.CompilerParams(
            dimension_semantics=("arbitrary", "arbitrary")),
    )(lab, idx, feat)
    loss = pl.pallas_call(
        _finalize_body,
        out_shape=jax.ShapeDtypeStruct((1, 1), jnp.float32),
    )(sums, counts)
    return loss.reshape(1)


# speedup vs baseline: 92.3230x; 2.4789x over previous
import jax
import jax.numpy as jnp
from jax import lax
from jax.experimental import pallas as pl
from jax.experimental.pallas import tpu as pltpu

N_CLASSES = 5
K = 32 * N_CLASSES
D = 192
HW = 384 * 384
B = 2
PB = 3072
NB = HW // PB
IGNORE_LB = 255


def _i0():
    return jnp.int32(0)


def _segsum_body(lab_ref, idx_ref, feat_ref, sums_ref, counts_ref):
    j = pl.program_id(1)

    @pl.when(j == 0)
    def _init():
        sums_ref[...] = jnp.zeros_like(sums_ref)
        counts_ref[...] = jnp.zeros_like(counts_ref)

    lab = lab_ref[0].reshape(1, PB)
    idx = idx_ref[0].reshape(1, PB)
    k2 = jnp.where(lab == IGNORE_LB, 0, lab * 32 + idx)
    rows = lax.broadcasted_iota(jnp.int32, (K, PB), 0)
    oh_t = (rows == jnp.broadcast_to(k2, (K, PB))).astype(jnp.bfloat16)
    feat = feat_ref[0].reshape(D, PB)
    hi = feat.astype(jnp.bfloat16)
    lo = (feat - hi.astype(jnp.float32)).astype(jnp.bfloat16)
    dn = (((1,), (1,)), ((), ()))
    acc = (lax.dot_general(hi, oh_t, dn, preferred_element_type=jnp.float32)
           + lax.dot_general(lo, oh_t, dn, preferred_element_type=jnp.float32))
    sums_ref[...] += acc[None]
    ones = jnp.ones((8, PB), jnp.bfloat16)
    cnt = lax.dot_general(ones, oh_t, dn, preferred_element_type=jnp.float32)
    counts_ref[...] += cnt[0:1][None]


def _finalize_body(sums_ref, counts_ref, out_ref):
    f32 = jnp.float32
    KK = (K, K)
    row = lax.broadcasted_iota(jnp.int32, KK, 0)
    col = lax.broadcasted_iota(jnp.int32, KK, 1)
    eye = row == col
    ks = lax.broadcasted_iota(jnp.int32, (1, K), 1)

    def _col(x, zero):
        return jnp.sum(jnp.where(eye, jnp.broadcast_to(x, KK), zero),
                       axis=1, keepdims=True, dtype=x.dtype)

    total = f32(0.0)
    n_valid = f32(0.0)

    for b in range(B):
        c2 = counts_ref[b]
        s2 = sums_ref[b]
        mx = jnp.max(jnp.where(c2 > 0, jnp.bitwise_and(ks, 31), 0))
        mx_safe = jnp.maximum(mx, 1)
        keyed = mx * jnp.right_shift(row, 5) + jnp.bitwise_and(row, 31)
        remap = (keyed == col).astype(f32)
        counts = jnp.dot(c2, remap, preferred_element_type=f32)
        sums = jnp.dot(s2, remap, preferred_element_type=f32)
        means = sums / jnp.maximum(counts, 1.0)

        present = counts > 0
        n_present = jnp.sum(present.astype(f32))

        P = jnp.zeros(KK, f32)
        for db in range(D // 8):
            md = means[db * 8:(db + 1) * 8, :]
            diff = jnp.abs(md[:, :, None] - md[:, None, :])
            P = P + jnp.sum(diff, axis=0)
        P = P * f32(1.0 / D)

        pk = jnp.where(present, ks, -1)
        k_last = jnp.max(pk)
        k_prev = jnp.max(jnp.where(ks == k_last, -1, pk))
        mxf = mx_safe.astype(f32)
        cls_f = jnp.floor((ks.astype(f32) - 1.0) / mxf)
        eff = jnp.where(ks == k_last,
                        jnp.floor((k_prev.astype(f32) - 1.0) / mxf),
                        cls_f)
        included = present & (counts >= 2.0) & (ks >= 1)
        incval = jnp.where(included, ks, -1)
        inc_col = _col(incval, 0)
        strict = row < col
        prev_idx = jnp.max(jnp.where(strict, jnp.broadcast_to(inc_col, KK), -1),
                           axis=0, keepdims=True)
        pidx = jnp.maximum(prev_idx, 0)
        eff_col = _col(eff, f32(0.0))
        gmat = row == jnp.broadcast_to(pidx, KK)
        prev_cls = jnp.sum(jnp.where(gmat, jnp.broadcast_to(eff_col, KK), 0.0),
                           axis=0, keepdims=True)
        prev_cls = jnp.where(prev_idx < 0, f32(-1e9), prev_cls)
        new_group = included & (eff != prev_cls)
        ng_col = _col(new_group.astype(f32), f32(0.0))
        lower = row <= col
        cums = jnp.sum(jnp.where(lower, jnp.broadcast_to(ng_col, KK), 0.0),
                       axis=0, keepdims=True)
        gid = jnp.where(included, cums - 1.0, f32(-1.0))
        memb = (row.astype(f32) == jnp.broadcast_to(gid, KK)).astype(f32)
        sizes = jnp.sum(memb, axis=1, keepdims=True)
        mp = jnp.dot(memb, P, preferred_element_type=f32)
        num = jnp.sum(mp * memb, axis=1, keepdims=True)
        ret = num / (jnp.maximum(sizes, 1.0) ** 2)
        ret = jnp.where(ret < 1.0, 0.5 * ret * ret, ret - 0.5)
        validg = (sizes > 0.0) & (n_present > 1.0)
        total = total + jnp.sum(jnp.where(validg, ret, 0.0))
        n_valid = n_valid + jnp.sum(validg.astype(f32))

    loss = jnp.where(n_valid > 0.0, total / jnp.maximum(n_valid, 1.0), 0.0)
    out_ref[...] = jnp.broadcast_to(loss * f32(1.0 / B), (1, 1))


def kernel(feature_out, labels, indexes):
    lab = labels.astype(jnp.int32)
    idx = indexes.astype(jnp.int32)
    sums, counts = pl.pallas_call(
        _segsum_body,
        grid=(B, NB),
        in_specs=[
            pl.BlockSpec((1, 8, 384), lambda b, j: (b, j, _i0())),
            pl.BlockSpec((1, 8, 384), lambda b, j: (b, j, _i0())),
            pl.BlockSpec((1, D, 8, 384), lambda b, j: (b, _i0(), j, _i0())),
        ],
        out_specs=[
            pl.BlockSpec((1, D, K), lambda b, j: (b, _i0(), _i0())),
            pl.BlockSpec((1, 1, K), lambda b, j: (b, _i0(), _i0())),
        ],
        out_shape=[
            jax.ShapeDtypeStruct((B, D, K), jnp.float32),
            jax.ShapeDtypeStruct((B, 1, K), jnp.float32),
        ],
        compiler_params=pltpu.CompilerParams(
            dimension_semantics=("arbitrary", "arbitrary")),
    )(lab, idx, feature_out)
    loss = pl.pallas_call(
        _finalize_body,
        out_shape=jax.ShapeDtypeStruct((1, 1), jnp.float32),
    )(sums, counts)
    return loss.reshape(1)


# speedup vs baseline: 111.0768x; 1.2031x over previous
import jax
import jax.numpy as jnp
from jax import lax
from jax.experimental import pallas as pl
from jax.experimental.pallas import tpu as pltpu

N_CLASSES = 5
K = 32 * N_CLASSES
D = 192
HW = 384 * 384
B = 2
PB = 3072
NB = HW // PB
IGNORE_LB = 255


def _i0():
    return jnp.int32(0)


def _segsum_body(lab_ref, idx_ref, feat_ref, sums_ref, counts_ref):
    j = pl.program_id(1)

    @pl.when(j == 0)
    def _init():
        sums_ref[...] = jnp.zeros_like(sums_ref)
        counts_ref[...] = jnp.zeros_like(counts_ref)

    lab = lab_ref[0].reshape(1, PB)
    idx = idx_ref[0].reshape(1, PB)
    k2 = jnp.where(lab == IGNORE_LB, 0, lab * 32 + idx)
    rows = lax.broadcasted_iota(jnp.int32, (K, PB), 0)
    oh_t = (rows == jnp.broadcast_to(k2, (K, PB))).astype(jnp.float32)
    feat = feat_ref[0].reshape(D, PB)
    dn = (((1,), (1,)), ((), ()))
    acc = lax.dot_general(feat, oh_t, dn, preferred_element_type=jnp.float32)
    sums_ref[...] += acc[None]
    counts_ref[...] += jnp.sum(oh_t, axis=1, keepdims=True)[None]


def _finalize_body(sums_ref, counts_ref, out_ref):
    f32 = jnp.float32
    KK = (K, K)
    row = lax.broadcasted_iota(jnp.int32, KK, 0)
    col = lax.broadcasted_iota(jnp.int32, KK, 1)
    eye = row == col
    ks = lax.broadcasted_iota(jnp.int32, (1, K), 1)

    def _col(x, zero):
        return jnp.sum(jnp.where(eye, jnp.broadcast_to(x, KK), zero),
                       axis=1, keepdims=True, dtype=x.dtype)

    total = f32(0.0)
    n_valid = f32(0.0)

    for b in range(B):
        c2col = counts_ref[b]
        c2 = jnp.sum(jnp.where(eye, jnp.broadcast_to(c2col, KK), 0.0),
                     axis=0, keepdims=True)
        s2 = sums_ref[b]
        mx = jnp.max(jnp.where(c2 > 0, jnp.bitwise_and(ks, 31), 0))
        mx_safe = jnp.maximum(mx, 1)
        keyed = mx * jnp.right_shift(row, 5) + jnp.bitwise_and(row, 31)
        remap = (keyed == col).astype(f32)
        counts = jnp.dot(c2, remap, preferred_element_type=f32)
        sums = jnp.dot(s2, remap, preferred_element_type=f32)
        means = sums / jnp.maximum(counts, 1.0)

        present = counts > 0
        n_present = jnp.sum(present.astype(f32))

        P = jnp.zeros(KK, f32)
        for db in range(D // 8):
            md = means[db * 8:(db + 1) * 8, :]
            diff = jnp.abs(md[:, :, None] - md[:, None, :])
            P = P + jnp.sum(diff, axis=0)
        P = P * f32(1.0 / D)

        pk = jnp.where(present, ks, -1)
        k_last = jnp.max(pk)
        k_prev = jnp.max(jnp.where(ks == k_last, -1, pk))
        mxf = mx_safe.astype(f32)
        cls_f = jnp.floor((ks.astype(f32) - 1.0) / mxf)
        eff = jnp.where(ks == k_last,
                        jnp.floor((k_prev.astype(f32) - 1.0) / mxf),
                        cls_f)
        included = present & (counts >= 2.0) & (ks >= 1)
        incval = jnp.where(included, ks, -1)
        inc_col = _col(incval, 0)
        strict = row < col
        prev_idx = jnp.max(jnp.where(strict, jnp.broadcast_to(inc_col, KK), -1),
                           axis=0, keepdims=True)
        pidx = jnp.maximum(prev_idx, 0)
        eff_col = _col(eff, f32(0.0))
        gmat = row == jnp.broadcast_to(pidx, KK)
        prev_cls = jnp.sum(jnp.where(gmat, jnp.broadcast_to(eff_col, KK), 0.0),
                           axis=0, keepdims=True)
        prev_cls = jnp.where(prev_idx < 0, f32(-1e9), prev_cls)
        new_group = included & (eff != prev_cls)
        ng_col = _col(new_group.astype(f32), f32(0.0))
        lower = row <= col
        cums = jnp.sum(jnp.where(lower, jnp.broadcast_to(ng_col, KK), 0.0),
                       axis=0, keepdims=True)
        gid = jnp.where(included, cums - 1.0, f32(-1.0))
        memb = (row.astype(f32) == jnp.broadcast_to(gid, KK)).astype(f32)
        sizes = jnp.sum(memb, axis=1, keepdims=True)
        mp = jnp.dot(memb, P, preferred_element_type=f32)
        num = jnp.sum(mp * memb, axis=1, keepdims=True)
        ret = num / (jnp.maximum(sizes, 1.0) ** 2)
        ret = jnp.where(ret < 1.0, 0.5 * ret * ret, ret - 0.5)
        validg = (sizes > 0.0) & (n_present > 1.0)
        total = total + jnp.sum(jnp.where(validg, ret, 0.0))
        n_valid = n_valid + jnp.sum(validg.astype(f32))

    loss = jnp.where(n_valid > 0.0, total / jnp.maximum(n_valid, 1.0), 0.0)
    out_ref[...] = jnp.broadcast_to(loss * f32(1.0 / B), (1, 1))


def kernel(feature_out, labels, indexes):
    lab = labels.astype(jnp.int32)
    idx = indexes.astype(jnp.int32)
    sums, counts = pl.pallas_call(
        _segsum_body,
        grid=(B, NB),
        in_specs=[
            pl.BlockSpec((1, 8, 384), lambda b, j: (b, j, _i0())),
            pl.BlockSpec((1, 8, 384), lambda b, j: (b, j, _i0())),
            pl.BlockSpec((1, D, 8, 384), lambda b, j: (b, _i0(), j, _i0())),
        ],
        out_specs=[
            pl.BlockSpec((1, D, K), lambda b, j: (b, _i0(), _i0())),
            pl.BlockSpec((1, K, 1), lambda b, j: (b, _i0(), _i0())),
        ],
        out_shape=[
            jax.ShapeDtypeStruct((B, D, K), jnp.float32),
            jax.ShapeDtypeStruct((B, K, 1), jnp.float32),
        ],
        compiler_params=pltpu.CompilerParams(
            dimension_semantics=("arbitrary", "arbitrary")),
    )(lab, idx, feature_out)
    loss = pl.pallas_call(
        _finalize_body,
        out_shape=jax.ShapeDtypeStruct((1, 1), jnp.float32),
    )(sums, counts)
    return loss.reshape(1)
